# Initial kernel scaffold; baseline (speedup 1.0000x reference)
import functools

import jax
import jax.numpy as jnp
from jax import lax
from jax.experimental import pallas as pl
from jax.experimental.pallas import tpu as pltpu
from jax.experimental.pallas import tpu_sc as plsc

N = 10000
E = 320000
B = 16
H = 64

NC, NS = 2, 16
NW = NC * NS
CH = 128
NCHUNK = E // CH
CPW = -(-NCHUNK // NW)
NPADC = CPW * NW
NPT = N // NS

BLK_E = 3200
GRID_E = E // BLK_E
BLK_N = 2000
GRID_N = N // BLK_N

_mesh = plsc.VectorSubcoreMesh(core_axis_name="c", subcore_axis_name="s")


def _dot(a, b):
    return jnp.dot(a, b, preferred_element_type=jnp.float32)


def _gather_body(pr, pc, rowp, colp, g3, idxr, idxc, bufa, bufb, sem):
    w = lax.axis_index("s") * NC + lax.axis_index("c")
    start = w * CPW
    n = jnp.minimum(CPW, NCHUNK - start)
    pltpu.sync_copy(rowp.at[pl.ds(start, CPW)], idxr)
    pltpu.sync_copy(colp.at[pl.ds(start, CPW)], idxc)

    def chunk(j, carry):
        c = start + j
        cpa = pltpu.async_copy(pr.at[idxr.at[j]], bufa, sem)
        cpb = pltpu.async_copy(pc.at[idxc.at[j]], bufb, sem)
        cpa.wait()
        cpb.wait()

        def addrow(i, carry2):
            for k in range(H // 16):
                sl = pl.ds(k * 16, 16)
                bufa[i, sl] = bufa[i, sl] + bufb[i, sl]
            return carry2

        lax.fori_loop(0, CH, addrow, 0)
        pltpu.sync_copy(bufa, g3.at[c])
        return carry

    lax.fori_loop(0, n, chunk, 0)


_gather_call = functools.partial(
    pl.kernel,
    out_type=jax.ShapeDtypeStruct((NCHUNK, CH, H), jnp.float32),
    mesh=_mesh,
    scratch_types=[
        pltpu.VMEM((CPW, CH), jnp.int32),
        pltpu.VMEM((CPW, CH), jnp.int32),
        pltpu.VMEM((CH, H), jnp.float32),
        pltpu.VMEM((CH, H), jnp.float32),
        pltpu.SemaphoreType.DMA,
    ],
)(_gather_body)


def _scatter_body(eo3, colp, parts, idxc, dbuf, zbuf, shared, sem):
    c_ax = lax.axis_index("c")
    s_ax = lax.axis_index("s")
    w = s_ax * NC + c_ax
    start = w * CPW
    n = jnp.minimum(CPW, NCHUNK - start)
    pltpu.sync_copy(colp.at[pl.ds(start, CPW)], idxc)

    def zrow(i, carry):
        zbuf[i, :] = jnp.zeros((16,), jnp.float32)
        return carry

    lax.fori_loop(0, NPT, zrow, 0)
    pltpu.sync_copy(zbuf, shared.at[pl.ds(s_ax * NPT, NPT)])
    plsc.subcore_barrier()

    def chunk(j, carry):
        c = start + j
        pltpu.sync_copy(eo3.at[c], dbuf)
        pltpu.sync_copy(dbuf, shared.at[idxc.at[j]], add=True)
        return carry

    lax.fori_loop(0, n, chunk, 0)
    plsc.subcore_barrier()
    pltpu.sync_copy(shared.at[pl.ds(s_ax * NPT, NPT)],
                    parts.at[c_ax, pl.ds(s_ax * NPT, NPT)])


_scatter_call = functools.partial(
    pl.kernel,
    out_type=jax.ShapeDtypeStruct((NC, N, 16), jnp.float32),
    mesh=_mesh,
    scratch_types=[
        pltpu.VMEM((CPW, CH), jnp.int32),
        pltpu.VMEM((CH, 16), jnp.float32),
        pltpu.VMEM((NPT, 16), jnp.float32),
        pltpu.VMEM_SHARED((N, 16), jnp.float32),
        pltpu.SemaphoreType.DMA,
    ],
)(_scatter_body)


def _prep_body(x_ref, wxr, wxc, wnx, u_ref, weu, eb1_ref, wnu, nb1_ref,
               pr_ref, pc_ref, pn_ref, ue_ref, un_ref):
    xb = x_ref[...]
    pr_ref[...] = _dot(xb, wxr[...])
    pc_ref[...] = _dot(xb, wxc[...])
    pn_ref[...] = _dot(xb, wnx[...])
    ub = u_ref[...]
    ue_ref[...] = _dot(ub, weu[...]) + eb1_ref[...]
    un_ref[...] = _dot(ub, wnu[...]) + nb1_ref[...]


def _prep_call(x, wxr, wxc, wnx, u, weu, eb1, wnu, nb1):
    full = lambda s: pl.BlockSpec(s, lambda i: (0, 0))
    return pl.pallas_call(
        _prep_body,
        grid=(GRID_N,),
        in_specs=[
            pl.BlockSpec((BLK_N, 128), lambda i: (i, 0)),
            full((128, H)), full((128, H)), full((128, H)),
            full((B, 32)), full((32, H)), full((1, H)), full((32, H)),
            full((1, H)),
        ],
        out_specs=[
            pl.BlockSpec((BLK_N, H), lambda i: (i, 0)),
            pl.BlockSpec((BLK_N, H), lambda i: (i, 0)),
            pl.BlockSpec((BLK_N, H), lambda i: (i, 0)),
            full((B, H)), full((B, H)),
        ],
        out_shape=[
            jax.ShapeDtypeStruct((N, H), jnp.float32),
            jax.ShapeDtypeStruct((N, H), jnp.float32),
            jax.ShapeDtypeStruct((N, H), jnp.float32),
            jax.ShapeDtypeStruct((B, H), jnp.float32),
            jax.ShapeDtypeStruct((B, H), jnp.float32),
        ],
    )(x, wxr, wxc, wnx, u, weu, eb1, wnu, nb1)


def _edge_body(g_ref, ea_ref, eidx_ref, ue_ref, wea, ew2, eb2_ref, eg_ref,
               ebeta_ref, eo_ref, se_ref, ce_ref):
    i = pl.program_id(0)
    e = eidx_ref[0, 0, :]
    oh = (e[:, None] == lax.broadcasted_iota(jnp.int32, (BLK_E, B), 1)
          ).astype(jnp.float32)
    h = g_ref[...] + _dot(ea_ref[...], wea[...]) + _dot(oh, ue_ref[...])
    h = jnp.maximum(h, 0.0)
    h2 = jnp.maximum(_dot(h, ew2[...]) + eb2_ref[...], 0.0)
    mu = jnp.mean(h2, axis=1, keepdims=True)
    d = h2 - mu
    var = jnp.mean(d * d, axis=1, keepdims=True)
    eo = d * lax.rsqrt(var + 1e-5) * eg_ref[...] + ebeta_ref[...]
    eo_ref[...] = eo

    @pl.when(i == 0)
    def _():
        se_ref[...] = jnp.zeros_like(se_ref)
        ce_ref[...] = jnp.zeros_like(ce_ref)

    ohT = (e[None, :] == lax.broadcasted_iota(jnp.int32, (B, BLK_E), 0)
           ).astype(jnp.float32)
    se_ref[...] += _dot(ohT, eo)
    ce_ref[...] += jnp.broadcast_to(jnp.sum(oh, axis=0)[:, None], (B, 8))


def _edge_call(g, ea, eidx3, ue, wea, ew2, eb2, eg, ebeta):
    full = lambda s: pl.BlockSpec(s, lambda i: (0,) * len(s))
    return pl.pallas_call(
        _edge_body,
        grid=(GRID_E,),
        in_specs=[
            pl.BlockSpec((BLK_E, H), lambda i: (i, 0)),
            pl.BlockSpec((BLK_E, 16), lambda i: (i, 0)),
            pl.BlockSpec((1, 1, BLK_E), lambda i: (i, 0, 0)),
            full((B, H)), full((16, H)), full((H, 16)), full((1, 16)),
            full((1, 16)), full((1, 16)),
        ],
        out_specs=[
            pl.BlockSpec((BLK_E, 16), lambda i: (i, 0)),
            full((B, 16)), full((B, 8)),
        ],
        out_shape=[
            jax.ShapeDtypeStruct((E, 16), jnp.float32),
            jax.ShapeDtypeStruct((B, 16), jnp.float32),
            jax.ShapeDtypeStruct((B, 8), jnp.float32),
        ],
    )(g, ea, eidx3, ue, wea, ew2, eb2, eg, ebeta)


def _node_body(pn_ref, s0_ref, s1_ref, vidx_ref, un_ref, wne, nw2, nb2_ref,
               ng_ref, nbeta_ref, xo_ref, sv_ref, cv_ref):
    i = pl.program_id(0)
    v = vidx_ref[0, 0, :]
    oh = (v[:, None] == lax.broadcasted_iota(jnp.int32, (BLK_N, B), 1)
          ).astype(jnp.float32)
    agg = s0_ref[...] + s1_ref[...]
    h = pn_ref[...] + _dot(agg, wne[...]) + _dot(oh, un_ref[...])
    h = jnp.maximum(h, 0.0)
    h2 = jnp.maximum(_dot(h, nw2[...]) + nb2_ref[...], 0.0)
    mu = jnp.mean(h2, axis=1, keepdims=True)
    d = h2 - mu
    var = jnp.mean(d * d, axis=1, keepdims=True)
    xo = d * lax.rsqrt(var + 1e-5) * ng_ref[...] + nbeta_ref[...]
    xo_ref[...] = xo

    @pl.when(i == 0)
    def _():
        sv_ref[...] = jnp.zeros_like(sv_ref)
        cv_ref[...] = jnp.zeros_like(cv_ref)

    ohT = (v[None, :] == lax.broadcasted_iota(jnp.int32, (B, BLK_N), 0)
           ).astype(jnp.float32)
    sv_ref[...] += _dot(ohT, xo)
    cv_ref[...] += jnp.broadcast_to(jnp.sum(oh, axis=0)[:, None], (B, 8))


def _node_call(pn, s0, s1, vidx3, un, wne, nw2, nb2, ng, nbeta):
    full = lambda s: pl.BlockSpec(s, lambda i: (0,) * len(s))
    return pl.pallas_call(
        _node_body,
        grid=(GRID_N,),
        in_specs=[
            pl.BlockSpec((BLK_N, H), lambda i: (i, 0)),
            pl.BlockSpec((BLK_N, 16), lambda i: (i, 0)),
            pl.BlockSpec((BLK_N, 16), lambda i: (i, 0)),
            pl.BlockSpec((1, 1, BLK_N), lambda i: (i, 0, 0)),
            full((B, H)), full((16, H)), full((H, 128)), full((1, 128)),
            full((1, 128)), full((1, 128)),
        ],
        out_specs=[
            pl.BlockSpec((BLK_N, 128), lambda i: (i, 0)),
            full((B, 128)), full((B, 8)),
        ],
        out_shape=[
            jax.ShapeDtypeStruct((N, 128), jnp.float32),
            jax.ShapeDtypeStruct((B, 128), jnp.float32),
            jax.ShapeDtypeStruct((B, 8), jnp.float32),
        ],
    )(pn, s0, s1, vidx3, un, wne, nw2, nb2, ng, nbeta)


def _global_body(u_ref, sv_ref, cv_ref, se_ref, ce_ref, gwu, gwv, gwe,
                 gb1_ref, gw2, gb2_ref, gg_ref, gbeta_ref, uo_ref):
    aggv = sv_ref[...] / jnp.maximum(cv_ref[:, 0:1], 1.0)
    agge = se_ref[...] / jnp.maximum(ce_ref[:, 0:1], 1.0)
    h = (_dot(u_ref[...], gwu[...]) + _dot(aggv, gwv[...])
         + _dot(agge, gwe[...]) + gb1_ref[...])
    h = jnp.maximum(h, 0.0)
    h2 = jnp.maximum(_dot(h, gw2[...]) + gb2_ref[...], 0.0)
    mu = jnp.mean(h2, axis=1, keepdims=True)
    d = h2 - mu
    var = jnp.mean(d * d, axis=1, keepdims=True)
    uo_ref[...] = d * lax.rsqrt(var + 1e-5) * gg_ref[...] + gbeta_ref[...]


def _global_call(u, sv, cv, se, ce, gwu, gwv, gwe, gb1, gw2, gb2, gg, gbeta):
    return pl.pallas_call(
        _global_body,
        out_shape=jax.ShapeDtypeStruct((B, 32), jnp.float32),
    )(u, sv, cv, se, ce, gwu, gwv, gwe, gb1, gw2, gb2, gg, gbeta)


def kernel(x, edge_index, edge_attr, u, v_indices, e_indices,
           eW1, eb1, eW2, eb2, eg, ebeta,
           nW1, nb1, nW2, nb2, ng, nbeta,
           gW1, gb1, gW2, gb2, gg, gbeta):
    row = edge_index[0]
    col = edge_index[1]

    wxr, wxc, wea, weu = eW1[:128], eW1[128:256], eW1[256:272], eW1[272:304]
    wnx, wne, wnu = nW1[:128], nW1[128:144], nW1[144:176]
    gwu, gwv, gwe = gW1[:32], gW1[32:160], gW1[160:176]

    r2 = lambda a: a.reshape(1, -1)

    pr, pc, pn, ue, un = _prep_call(x, wxr, wxc, wnx, u, weu, r2(eb1),
                                    wnu, r2(nb1))

    pad = jnp.zeros((NPADC * CH - E,), jnp.int32)
    rowp = jnp.concatenate([row, pad]).reshape(NPADC, CH)
    colp = jnp.concatenate([col, pad]).reshape(NPADC, CH)

    g3 = _gather_call(pr, pc, rowp, colp)
    g = g3.reshape(E, H)

    eidx3 = e_indices.reshape(GRID_E, 1, BLK_E)
    eo, se, ce = _edge_call(g, edge_attr, eidx3, ue, wea, eW2, r2(eb2),
                            r2(eg), r2(ebeta))

    parts = _scatter_call(eo.reshape(NCHUNK, CH, 16), colp)

    vidx3 = v_indices.reshape(GRID_N, 1, BLK_N)
    xo, sv, cv = _node_call(pn, parts[0], parts[1], vidx3, un, wne, nW2,
                            r2(nb2), r2(ng), r2(nbeta))

    uo = _global_call(u, sv, cv, se, ce, gwu, gwv, gwe, r2(gb1), gW2,
                      r2(gb2), r2(gg), r2(gbeta))
    return (xo, eo, uo)


# speedup vs baseline: 4.7192x; 4.7192x over previous
import functools

import jax
import jax.numpy as jnp
from jax import lax
from jax.experimental import pallas as pl
from jax.experimental.pallas import tpu as pltpu
from jax.experimental.pallas import tpu_sc as plsc

N = 10000
E = 320000
B = 16
H = 64

NC, NS = 2, 16
NW = NC * NS
CH = 128
NCHUNK = E // CH
CPW = 80
NPADC = CPW * NW
NPAD_N = 10240
NPT = NPAD_N // NS

BLK_E = 3200
GRID_E = E // BLK_E
BLK_N = 2000
GRID_N = N // BLK_N

@functools.cache
def _sc_mesh():
    return plsc.VectorSubcoreMesh(core_axis_name="c", subcore_axis_name="s",
                                  num_cores=NC, num_subcores=NS)


def _dot(a, b):
    return jnp.dot(a, b, preferred_element_type=jnp.float32)


def _gather_body(pr, pc, rowp, colp, g3, idxr, idxc, bufa, bufb, sem):
    w = lax.axis_index("s") * NC + lax.axis_index("c")
    start = w * CPW
    n = jnp.minimum(CPW, NCHUNK - start)
    pltpu.sync_copy(rowp.at[pl.ds(start, CPW)], idxr)
    pltpu.sync_copy(colp.at[pl.ds(start, CPW)], idxc)

    def chunk(j, carry):
        c = start + j
        cpa = pltpu.async_copy(pr.at[idxr.at[j]], bufa, sem)
        cpb = pltpu.async_copy(pc.at[idxc.at[j]], bufb, sem)
        cpa.wait()
        cpb.wait()

        def addrow(i, carry2):
            for k in range(H // 16):
                sl = pl.ds(k * 16, 16)
                bufa[i, sl] = bufa[i, sl] + bufb[i, sl]
            return carry2

        lax.fori_loop(0, CH, addrow, 0)
        pltpu.sync_copy(bufa, g3.at[c])
        return carry

    lax.fori_loop(0, n, chunk, 0)


@functools.cache
def _gather_kernel():
    return pl.kernel(
        _gather_body,
        out_type=jax.ShapeDtypeStruct((NCHUNK, CH, H), jnp.float32),
        mesh=_sc_mesh(),
        compiler_params=pltpu.CompilerParams(use_tc_tiling_on_sc=False),
        scratch_types=[
            pltpu.VMEM((CPW, CH), jnp.int32),
            pltpu.VMEM((CPW, CH), jnp.int32),
            pltpu.VMEM((CH, H), jnp.float32),
            pltpu.VMEM((CH, H), jnp.float32),
            pltpu.SemaphoreType.DMA,
        ],
    )


def _gather_call(pr, pc, rowp, colp):
    return _gather_kernel()(pr, pc, rowp, colp)


def _scatter_body(eo3, colp, parts, idxc, dbuf, zbuf, shared, sem):
    c_ax = lax.axis_index("c")
    s_ax = lax.axis_index("s")
    w = s_ax * NC + c_ax
    start = w * CPW
    n = jnp.minimum(CPW, NCHUNK - start)
    pltpu.sync_copy(colp.at[pl.ds(start, CPW)], idxc)

    def zrow(i, carry):
        zbuf[i, :] = jnp.zeros((16,), jnp.float32)
        return carry

    lax.fori_loop(0, NPT, zrow, 0)
    pltpu.sync_copy(zbuf, shared.at[pl.ds(s_ax * NPT, NPT)])
    plsc.subcore_barrier()

    def chunk(j, carry):
        c = start + j
        pltpu.sync_copy(eo3.at[c], dbuf)
        pltpu.sync_copy(dbuf, shared.at[idxc.at[j]], add=True)
        return carry

    lax.fori_loop(0, n, chunk, 0)
    plsc.subcore_barrier()
    pltpu.sync_copy(shared.at[pl.ds(s_ax * NPT, NPT)],
                    parts.at[c_ax, pl.ds(s_ax * NPT, NPT)])


@functools.cache
def _scatter_kernel():
    return pl.kernel(
        _scatter_body,
        out_type=jax.ShapeDtypeStruct((NC, NPAD_N, 16), jnp.float32),
        mesh=_sc_mesh(),
        compiler_params=pltpu.CompilerParams(use_tc_tiling_on_sc=False),
        scratch_types=[
            pltpu.VMEM((CPW, CH), jnp.int32),
            pltpu.VMEM((CH, 16), jnp.float32),
            pltpu.VMEM((NPT, 16), jnp.float32),
            pltpu.VMEM_SHARED((NPAD_N, 16), jnp.float32),
            pltpu.SemaphoreType.DMA,
        ],
    )


def _scatter_call(eo3, colp):
    return _scatter_kernel()(eo3, colp)


def _prep_body(x_ref, wxr, wxc, wnx, u_ref, weu, eb1_ref, wnu, nb1_ref,
               pr_ref, pc_ref, pn_ref, ue_ref, un_ref):
    xb = x_ref[...]
    pr_ref[...] = _dot(xb, wxr[...])
    pc_ref[...] = _dot(xb, wxc[...])
    pn_ref[...] = _dot(xb, wnx[...])
    ub = u_ref[...]
    ue_ref[...] = _dot(ub, weu[...]) + eb1_ref[...]
    un_ref[...] = _dot(ub, wnu[...]) + nb1_ref[...]


def _prep_call(x, wxr, wxc, wnx, u, weu, eb1, wnu, nb1):
    full = lambda s: pl.BlockSpec(s, lambda i: (0, 0))
    return pl.pallas_call(
        _prep_body,
        grid=(GRID_N,),
        in_specs=[
            pl.BlockSpec((BLK_N, 128), lambda i: (i, 0)),
            full((128, H)), full((128, H)), full((128, H)),
            full((B, 32)), full((32, H)), full((1, H)), full((32, H)),
            full((1, H)),
        ],
        out_specs=[
            pl.BlockSpec((BLK_N, H), lambda i: (i, 0)),
            pl.BlockSpec((BLK_N, H), lambda i: (i, 0)),
            pl.BlockSpec((BLK_N, H), lambda i: (i, 0)),
            full((B, H)), full((B, H)),
        ],
        out_shape=[
            jax.ShapeDtypeStruct((N, H), jnp.float32),
            jax.ShapeDtypeStruct((N, H), jnp.float32),
            jax.ShapeDtypeStruct((N, H), jnp.float32),
            jax.ShapeDtypeStruct((B, H), jnp.float32),
            jax.ShapeDtypeStruct((B, H), jnp.float32),
        ],
    )(x, wxr, wxc, wnx, u, weu, eb1, wnu, nb1)


def _edge_body(g_ref, ea_ref, eidx_ref, ue_ref, wea, ew2, eb2_ref, eg_ref,
               ebeta_ref, eo_ref, se_ref, ce_ref):
    i = pl.program_id(0)
    e = eidx_ref[0, 0, :]
    oh = (e[:, None] == lax.broadcasted_iota(jnp.int32, (BLK_E, B), 1)
          ).astype(jnp.float32)
    h = g_ref[...] + _dot(ea_ref[...], wea[...]) + _dot(oh, ue_ref[...])
    h = jnp.maximum(h, 0.0)
    h2 = jnp.maximum(_dot(h, ew2[...]) + eb2_ref[...], 0.0)
    mu = jnp.mean(h2, axis=1, keepdims=True)
    d = h2 - mu
    var = jnp.mean(d * d, axis=1, keepdims=True)
    eo = d * lax.rsqrt(var + 1e-5) * eg_ref[...] + ebeta_ref[...]
    eo_ref[...] = eo

    @pl.when(i == 0)
    def _():
        se_ref[...] = jnp.zeros_like(se_ref)
        ce_ref[...] = jnp.zeros_like(ce_ref)

    ohT = (e[None, :] == lax.broadcasted_iota(jnp.int32, (B, BLK_E), 0)
           ).astype(jnp.float32)
    se_ref[...] += _dot(ohT, eo)
    ce_ref[...] += jnp.broadcast_to(jnp.sum(oh, axis=0)[:, None], (B, 8))


def _edge_call(g, ea, eidx3, ue, wea, ew2, eb2, eg, ebeta):
    full = lambda s: pl.BlockSpec(s, lambda i: (0,) * len(s))
    return pl.pallas_call(
        _edge_body,
        grid=(GRID_E,),
        in_specs=[
            pl.BlockSpec((BLK_E, H), lambda i: (i, 0)),
            pl.BlockSpec((BLK_E, 16), lambda i: (i, 0)),
            pl.BlockSpec((1, 1, BLK_E), lambda i: (i, 0, 0)),
            full((B, H)), full((16, H)), full((H, 16)), full((1, 16)),
            full((1, 16)), full((1, 16)),
        ],
        out_specs=[
            pl.BlockSpec((BLK_E, 16), lambda i: (i, 0)),
            full((B, 16)), full((B, 8)),
        ],
        out_shape=[
            jax.ShapeDtypeStruct((E, 16), jnp.float32),
            jax.ShapeDtypeStruct((B, 16), jnp.float32),
            jax.ShapeDtypeStruct((B, 8), jnp.float32),
        ],
    )(g, ea, eidx3, ue, wea, ew2, eb2, eg, ebeta)


def _node_body(pn_ref, s0_ref, s1_ref, vidx_ref, un_ref, wne, nw2, nb2_ref,
               ng_ref, nbeta_ref, xo_ref, sv_ref, cv_ref):
    i = pl.program_id(0)
    v = vidx_ref[0, 0, :]
    oh = (v[:, None] == lax.broadcasted_iota(jnp.int32, (BLK_N, B), 1)
          ).astype(jnp.float32)
    agg = s0_ref[...] + s1_ref[...]
    h = pn_ref[...] + _dot(agg, wne[...]) + _dot(oh, un_ref[...])
    h = jnp.maximum(h, 0.0)
    h2 = jnp.maximum(_dot(h, nw2[...]) + nb2_ref[...], 0.0)
    mu = jnp.mean(h2, axis=1, keepdims=True)
    d = h2 - mu
    var = jnp.mean(d * d, axis=1, keepdims=True)
    xo = d * lax.rsqrt(var + 1e-5) * ng_ref[...] + nbeta_ref[...]
    xo_ref[...] = xo

    @pl.when(i == 0)
    def _():
        sv_ref[...] = jnp.zeros_like(sv_ref)
        cv_ref[...] = jnp.zeros_like(cv_ref)

    ohT = (v[None, :] == lax.broadcasted_iota(jnp.int32, (B, BLK_N), 0)
           ).astype(jnp.float32)
    sv_ref[...] += _dot(ohT, xo)
    cv_ref[...] += jnp.broadcast_to(jnp.sum(oh, axis=0)[:, None], (B, 8))


def _node_call(pn, s0, s1, vidx3, un, wne, nw2, nb2, ng, nbeta):
    full = lambda s: pl.BlockSpec(s, lambda i: (0,) * len(s))
    return pl.pallas_call(
        _node_body,
        grid=(GRID_N,),
        in_specs=[
            pl.BlockSpec((BLK_N, H), lambda i: (i, 0)),
            pl.BlockSpec((BLK_N, 16), lambda i: (i, 0)),
            pl.BlockSpec((BLK_N, 16), lambda i: (i, 0)),
            pl.BlockSpec((1, 1, BLK_N), lambda i: (i, 0, 0)),
            full((B, H)), full((16, H)), full((H, 128)), full((1, 128)),
            full((1, 128)), full((1, 128)),
        ],
        out_specs=[
            pl.BlockSpec((BLK_N, 128), lambda i: (i, 0)),
            full((B, 128)), full((B, 8)),
        ],
        out_shape=[
            jax.ShapeDtypeStruct((N, 128), jnp.float32),
            jax.ShapeDtypeStruct((B, 128), jnp.float32),
            jax.ShapeDtypeStruct((B, 8), jnp.float32),
        ],
    )(pn, s0, s1, vidx3, un, wne, nw2, nb2, ng, nbeta)


def _global_body(u_ref, sv_ref, cv_ref, se_ref, ce_ref, gwu, gwv, gwe,
                 gb1_ref, gw2, gb2_ref, gg_ref, gbeta_ref, uo_ref):
    aggv = sv_ref[...] / jnp.maximum(cv_ref[:, 0:1], 1.0)
    agge = se_ref[...] / jnp.maximum(ce_ref[:, 0:1], 1.0)
    h = (_dot(u_ref[...], gwu[...]) + _dot(aggv, gwv[...])
         + _dot(agge, gwe[...]) + gb1_ref[...])
    h = jnp.maximum(h, 0.0)
    h2 = jnp.maximum(_dot(h, gw2[...]) + gb2_ref[...], 0.0)
    mu = jnp.mean(h2, axis=1, keepdims=True)
    d = h2 - mu
    var = jnp.mean(d * d, axis=1, keepdims=True)
    uo_ref[...] = d * lax.rsqrt(var + 1e-5) * gg_ref[...] + gbeta_ref[...]


def _global_call(u, sv, cv, se, ce, gwu, gwv, gwe, gb1, gw2, gb2, gg, gbeta):
    return pl.pallas_call(
        _global_body,
        out_shape=jax.ShapeDtypeStruct((B, 32), jnp.float32),
    )(u, sv, cv, se, ce, gwu, gwv, gwe, gb1, gw2, gb2, gg, gbeta)


def kernel(x, edge_index, edge_attr, u, v_indices, e_indices,
           eW1, eb1, eW2, eb2, eg, ebeta,
           nW1, nb1, nW2, nb2, ng, nbeta,
           gW1, gb1, gW2, gb2, gg, gbeta):
    row = edge_index[0]
    col = edge_index[1]

    wxr, wxc, wea, weu = eW1[:128], eW1[128:256], eW1[256:272], eW1[272:304]
    wnx, wne, wnu = nW1[:128], nW1[128:144], nW1[144:176]
    gwu, gwv, gwe = gW1[:32], gW1[32:160], gW1[160:176]

    r2 = lambda a: a.reshape(1, -1)

    pr, pc, pn, ue, un = _prep_call(x, wxr, wxc, wnx, u, weu, r2(eb1),
                                    wnu, r2(nb1))

    pad = jnp.zeros((NPADC * CH - E,), jnp.int32)
    rowp = jnp.concatenate([row, pad]).reshape(NPADC, CH)
    colp = jnp.concatenate([col, pad]).reshape(NPADC, CH)

    g3 = _gather_call(pr, pc, rowp, colp)
    g = g3.reshape(E, H)

    eidx3 = e_indices.reshape(GRID_E, 1, BLK_E)
    eo, se, ce = _edge_call(g, edge_attr, eidx3, ue, wea, eW2, r2(eb2),
                            r2(eg), r2(ebeta))

    parts = _scatter_call(eo.reshape(NCHUNK, CH, 16), colp)

    vidx3 = v_indices.reshape(GRID_N, 1, BLK_N)
    xo, sv, cv = _node_call(pn, parts[0], parts[1], vidx3, un, wne, nW2,
                            r2(nb2), r2(ng), r2(nbeta))

    uo = _global_call(u, sv, cv, se, ce, gwu, gwv, gwe, r2(gb1), gW2,
                      r2(gb2), r2(gg), r2(gbeta))
    return (xo, eo, uo)
